# Initial kernel scaffold; baseline (speedup 1.0000x reference)
import jax
import jax.numpy as jnp
from jax.experimental import pallas as pl


def kernel(hidden_states, Wg, W1, W2, W3):
    raise NotImplementedError("write your pallas kernel here")


# speedup vs baseline: 2.0037x; 2.0037x over previous
import functools

import jax
import jax.numpy as jnp
from jax.experimental import pallas as pl
from jax.experimental.pallas import tpu as pltpu
from jax.experimental.pallas import tpu_sc as plsc

NUM_EXPERTS = 8
TOP_K = 2
HIDDEN = 1024
FFN = 4096

TB = 256
M = 256
FB = 512
NF = FFN // FB

SC_CORES = 2
SC_SUBCORES = 16
SC_WORKERS = SC_CORES * SC_SUBCORES
SC_CHUNK = 32


def _router_body(x_ref, wg_ref, sel0_ref, sel1_ref, rank0_ref, rank1_ref,
                 w0_ref, w1_ref, counts_ref, probsum_ref, carry_ref, psum_ref):
    b = pl.program_id(0)

    @pl.when(b == 0)
    def _():
        carry_ref[...] = jnp.zeros_like(carry_ref)
        psum_ref[...] = jnp.zeros_like(psum_ref)

    xb = x_ref[...]
    logits = jax.lax.dot_general(
        xb, wg_ref[...], (((1,), (1,)), ((), ())),
        preferred_element_type=jnp.float32)
    mx = jnp.max(logits, axis=1, keepdims=True)
    ex = jnp.exp(logits - mx)
    probs = ex / jnp.sum(ex, axis=1, keepdims=True)

    lane = jax.lax.broadcasted_iota(jnp.int32, (TB, NUM_EXPERTS), 1)
    p0 = jnp.max(probs, axis=1, keepdims=True)
    e0 = jnp.min(jnp.where(probs == p0, lane, NUM_EXPERTS), axis=1,
                 keepdims=True)
    oh0 = (lane == e0)
    probs2 = jnp.where(oh0, -1.0, probs)
    p1 = jnp.max(probs2, axis=1, keepdims=True)
    e1 = jnp.min(jnp.where(probs2 == p1, lane, NUM_EXPERTS), axis=1,
                 keepdims=True)
    oh1 = (lane == e1)

    s = p0 + p1
    w0_ref[...] = p0 / s
    w1_ref[...] = p1 / s
    sel0_ref[...] = e0
    sel1_ref[...] = e1

    oh0f = oh0.astype(jnp.float32)
    oh1f = oh1.astype(jnp.float32)
    r_i = jax.lax.broadcasted_iota(jnp.int32, (TB, TB), 0)
    c_i = jax.lax.broadcasted_iota(jnp.int32, (TB, TB), 1)
    tri = (r_i >= c_i).astype(jnp.float32)
    cum0 = jax.lax.dot_general(tri, oh0f, (((1,), (0,)), ((), ())),
                               preferred_element_type=jnp.float32)
    cum1 = jax.lax.dot_general(tri, oh1f, (((1,), (0,)), ((), ())),
                               preferred_element_type=jnp.float32)
    carry = carry_ref[...]
    colsum0 = jnp.sum(oh0f, axis=0, keepdims=True)
    colsum1 = jnp.sum(oh1f, axis=0, keepdims=True)
    rank0 = jnp.sum(oh0f * (cum0 + carry), axis=1, keepdims=True) - 1.0
    rank1 = jnp.sum(oh1f * (cum1 + carry + colsum0), axis=1, keepdims=True) - 1.0
    rank0_ref[...] = rank0.astype(jnp.int32)
    rank1_ref[...] = rank1.astype(jnp.int32)

    carry_ref[...] = carry + colsum0 + colsum1
    psum_ref[...] = psum_ref[...] + jnp.sum(probs, axis=0, keepdims=True)
    counts_ref[...] = carry_ref[...]
    probsum_ref[...] = psum_ref[...]


def _router(x, Wg):
    T = x.shape[0]
    nb = T // TB
    out_shapes = (
        jax.ShapeDtypeStruct((T, 1), jnp.int32),
        jax.ShapeDtypeStruct((T, 1), jnp.int32),
        jax.ShapeDtypeStruct((T, 1), jnp.int32),
        jax.ShapeDtypeStruct((T, 1), jnp.int32),
        jax.ShapeDtypeStruct((T, 1), jnp.float32),
        jax.ShapeDtypeStruct((T, 1), jnp.float32),
        jax.ShapeDtypeStruct((1, NUM_EXPERTS), jnp.float32),
        jax.ShapeDtypeStruct((1, NUM_EXPERTS), jnp.float32),
    )
    col = pl.BlockSpec((TB, 1), lambda b: (b, 0))
    small = pl.BlockSpec((1, NUM_EXPERTS), lambda b: (0, 0))
    return pl.pallas_call(
        _router_body,
        grid=(nb,),
        in_specs=[
            pl.BlockSpec((TB, HIDDEN), lambda b: (b, 0)),
            pl.BlockSpec((NUM_EXPERTS, HIDDEN), lambda b: (0, 0)),
        ],
        out_specs=(col, col, col, col, col, col, small, small),
        out_shape=out_shapes,
        scratch_shapes=[
            pltpu.VMEM((1, NUM_EXPERTS), jnp.float32),
            pltpu.VMEM((1, NUM_EXPERTS), jnp.float32),
        ],
    )(x, Wg)


def _pos_loss_body(sel0_ref, sel1_ref, rank0_ref, rank1_ref, counts_ref,
                   probsum_ref, pos0_ref, pos1_ref, loss_ref):
    counts = counts_ref[...]
    e_i = jax.lax.broadcasted_iota(jnp.int32, (NUM_EXPERTS, NUM_EXPERTS), 0)
    f_i = jax.lax.broadcasted_iota(jnp.int32, (NUM_EXPERTS, NUM_EXPERTS), 1)
    lower = (e_i < f_i).astype(jnp.float32)
    off = jax.lax.dot_general(counts, lower, (((1,), (0,)), ((), ())),
                              preferred_element_type=jnp.float32)

    def gather_off(sel):
        acc = jnp.zeros(sel.shape, jnp.float32)
        for e in range(NUM_EXPERTS):
            acc = jnp.where(sel == e, off[0, e], acc)
        return acc.astype(jnp.int32)

    pos0_ref[...] = rank0_ref[...] + gather_off(sel0_ref[...])
    pos1_ref[...] = rank1_ref[...] + gather_off(sel1_ref[...])

    t_total = jnp.sum(counts) / TOP_K
    loss_ref[...] = (jnp.sum(counts * probsum_ref[...])
                     / (t_total * t_total)).reshape(1, 1)


def _pos_loss(sel0, sel1, rank0, rank1, counts, probsum):
    T = sel0.shape[0]
    return pl.pallas_call(
        _pos_loss_body,
        out_shape=(
            jax.ShapeDtypeStruct((T, 1), jnp.int32),
            jax.ShapeDtypeStruct((T, 1), jnp.int32),
            jax.ShapeDtypeStruct((1, 1), jnp.float32),
        ),
    )(sel0, sel1, rank0, rank1, counts, probsum)


def _sc_dispatch(x, pos0, pos1):
    T, H = x.shape
    tw = T // SC_WORKERS
    mesh = plsc.VectorSubcoreMesh(core_axis_name="c", subcore_axis_name="s",
                                  num_cores=SC_CORES, num_subcores=SC_SUBCORES)

    @functools.partial(
        pl.kernel,
        out_type=jax.ShapeDtypeStruct((TOP_K * T, H), x.dtype),
        mesh=mesh,
        scratch_types=[
            pltpu.VMEM((SC_CHUNK, H), x.dtype),
            pltpu.VMEM((SC_CHUNK,), jnp.int32),
            pltpu.VMEM((SC_CHUNK,), jnp.int32),
        ],
    )
    def k(x_hbm, p0_hbm, p1_hbm, xg_hbm, rows_v, i0_v, i1_v):
        wid = jax.lax.axis_index("s") * SC_CORES + jax.lax.axis_index("c")
        base = wid * tw
        for j in range(tw // SC_CHUNK):
            b = base + j * SC_CHUNK
            pltpu.sync_copy(x_hbm.at[pl.ds(b, SC_CHUNK)], rows_v)
            pltpu.sync_copy(p0_hbm.at[pl.ds(b, SC_CHUNK)], i0_v)
            pltpu.sync_copy(p1_hbm.at[pl.ds(b, SC_CHUNK)], i1_v)
            pltpu.sync_copy(rows_v, xg_hbm.at[i0_v])
            pltpu.sync_copy(rows_v, xg_hbm.at[i1_v])

    return k(x, pos0, pos1)


def _sc_combine_gather(yg, pos0, pos1):
    T = pos0.shape[0]
    H = yg.shape[1]
    tw = T // SC_WORKERS
    mesh = plsc.VectorSubcoreMesh(core_axis_name="c", subcore_axis_name="s",
                                  num_cores=SC_CORES, num_subcores=SC_SUBCORES)

    @functools.partial(
        pl.kernel,
        out_type=(jax.ShapeDtypeStruct((T, H), yg.dtype),
                  jax.ShapeDtypeStruct((T, H), yg.dtype)),
        mesh=mesh,
        scratch_types=[
            pltpu.VMEM((SC_CHUNK, H), yg.dtype),
            pltpu.VMEM((SC_CHUNK,), jnp.int32),
            pltpu.SemaphoreType.DMA,
        ],
    )
    def k(yg_hbm, p0_hbm, p1_hbm, y0_hbm, y1_hbm, rows_v, i_v, sem):
        wid = jax.lax.axis_index("s") * SC_CORES + jax.lax.axis_index("c")
        base = wid * tw
        for j in range(tw // SC_CHUNK):
            b = base + j * SC_CHUNK
            pltpu.sync_copy(p0_hbm.at[pl.ds(b, SC_CHUNK)], i_v)
            pltpu.async_copy(yg_hbm.at[i_v], rows_v, sem).wait()
            pltpu.sync_copy(rows_v, y0_hbm.at[pl.ds(b, SC_CHUNK)])
            pltpu.sync_copy(p1_hbm.at[pl.ds(b, SC_CHUNK)], i_v)
            pltpu.async_copy(yg_hbm.at[i_v], rows_v, sem).wait()
            pltpu.sync_copy(rows_v, y1_hbm.at[pl.ds(b, SC_CHUNK)])

    return k(yg, pos0, pos1)


def _ffn_body(vt_ref, ve_ref, vlo_ref, vhi_ref, xg_ref, w1_ref, w3_ref,
              w2_ref, out_ref):
    g = pl.program_id(0)
    lo = vlo_ref[g]
    hi = vhi_ref[g]
    prev_t = vt_ref[jnp.maximum(g - 1, 0)]
    first = jnp.logical_or(g == 0, vt_ref[g] != prev_t)

    @pl.when(hi > lo)
    def _():
        xb = xg_ref[...].astype(jnp.bfloat16)
        acc = jnp.zeros((M, HIDDEN), jnp.float32)
        for nf in range(NF):
            w1c = w1_ref[0, nf * FB:(nf + 1) * FB, :]
            w3c = w3_ref[0, nf * FB:(nf + 1) * FB, :]
            a = jax.lax.dot_general(xb, w1c, (((1,), (1,)), ((), ())),
                                    preferred_element_type=jnp.float32)
            bb = jax.lax.dot_general(xb, w3c, (((1,), (1,)), ((), ())),
                                     preferred_element_type=jnp.float32)
            hh = (a * jax.lax.logistic(a) * bb).astype(jnp.bfloat16)
            w2c = w2_ref[0, :, nf * FB:(nf + 1) * FB]
            acc = acc + jax.lax.dot_general(
                hh, w2c, (((1,), (1,)), ((), ())),
                preferred_element_type=jnp.float32)
        rows = jax.lax.broadcasted_iota(jnp.int32, (M, 1), 0)
        maskv = jnp.logical_and(rows >= lo, rows < hi)
        contrib = jnp.where(maskv, acc, 0.0)

        @pl.when(first)
        def _():
            out_ref[...] = contrib

        @pl.when(jnp.logical_not(first))
        def _():
            out_ref[...] = out_ref[...] + contrib


def _ffn(xg, W1b, W3b, W2b, vt, ve, vlo, vhi, num_visits):
    R = xg.shape[0]
    grid_spec = pltpu.PrefetchScalarGridSpec(
        num_scalar_prefetch=4,
        grid=(num_visits,),
        in_specs=[
            pl.BlockSpec((M, HIDDEN), lambda g, vt, ve, vlo, vhi: (vt[g], 0)),
            pl.BlockSpec((1, FFN, HIDDEN),
                         lambda g, vt, ve, vlo, vhi: (ve[g], 0, 0)),
            pl.BlockSpec((1, FFN, HIDDEN),
                         lambda g, vt, ve, vlo, vhi: (ve[g], 0, 0)),
            pl.BlockSpec((1, HIDDEN, FFN),
                         lambda g, vt, ve, vlo, vhi: (ve[g], 0, 0)),
        ],
        out_specs=pl.BlockSpec((M, HIDDEN),
                               lambda g, vt, ve, vlo, vhi: (vt[g], 0)),
    )
    return pl.pallas_call(
        _ffn_body,
        grid_spec=grid_spec,
        out_shape=jax.ShapeDtypeStruct((R, HIDDEN), jnp.float32),
    )(vt, ve, vlo, vhi, xg, W1b, W3b, W2b)


def _combine_body(y0_ref, y1_ref, w0_ref, w1_ref, out_ref):
    out_ref[...] = y0_ref[...] * w0_ref[...] + y1_ref[...] * w1_ref[...]


def _combine(y0, y1, w0, w1):
    T, H = y0.shape
    blk = 512
    return pl.pallas_call(
        _combine_body,
        grid=(T // blk,),
        in_specs=[
            pl.BlockSpec((blk, H), lambda b: (b, 0)),
            pl.BlockSpec((blk, H), lambda b: (b, 0)),
            pl.BlockSpec((blk, 1), lambda b: (b, 0)),
            pl.BlockSpec((blk, 1), lambda b: (b, 0)),
        ],
        out_specs=pl.BlockSpec((blk, H), lambda b: (b, 0)),
        out_shape=jax.ShapeDtypeStruct((T, H), jnp.float32),
    )(y0, y1, w0, w1)


def kernel(hidden_states, Wg, W1, W2, W3):
    B, S, H = hidden_states.shape
    T = B * S
    R = TOP_K * T
    NT = R // M
    G = NT + NUM_EXPERTS - 1

    x = hidden_states.reshape(T, H)

    sel0, sel1, rank0, rank1, w0, w1, counts, probsum = _router(x, Wg)
    pos0c, pos1c, loss2d = _pos_loss(sel0, sel1, rank0, rank1, counts, probsum)
    pos0 = pos0c.reshape(T)
    pos1 = pos1c.reshape(T)

    counts_i = counts.reshape(NUM_EXPERTS).astype(jnp.int32)
    offs = jnp.concatenate([jnp.zeros((1,), jnp.int32), jnp.cumsum(counts_i)])
    tile_lo = (jnp.arange(NT, dtype=jnp.int32) * M)
    ov_lo = jnp.maximum(offs[:-1][None, :], tile_lo[:, None])
    ov_hi = jnp.minimum(offs[1:][None, :], (tile_lo + M)[:, None])
    valid = (ov_hi > ov_lo).reshape(-1)
    vrank = jnp.cumsum(valid.astype(jnp.int32)) - valid.astype(jnp.int32)
    V = jnp.sum(valid.astype(jnp.int32))
    dest = jnp.where(valid, vrank, G)
    ar = jnp.arange(NT * NUM_EXPERTS, dtype=jnp.int32)
    mm = ar // NUM_EXPERTS
    ee = ar % NUM_EXPERTS
    zeros_g = jnp.zeros((G,), jnp.int32)
    vt = zeros_g.at[dest].set(mm, mode="drop")
    ve = zeros_g.at[dest].set(ee, mode="drop")
    vlo = zeros_g.at[dest].set((ov_lo - tile_lo[:, None]).reshape(-1),
                               mode="drop")
    vhi = zeros_g.at[dest].set((ov_hi - tile_lo[:, None]).reshape(-1),
                               mode="drop")
    idxg = jnp.arange(G, dtype=jnp.int32)
    vt = jnp.where(idxg < V, vt, jnp.take(vt, V - 1))
    ve = jnp.where(idxg < V, ve, jnp.take(ve, V - 1))
    vlo = jnp.where(idxg < V, vlo, 0)
    vhi = jnp.where(idxg < V, vhi, 0)

    xg = _sc_dispatch(x, pos0, pos1)
    W1b = W1.astype(jnp.bfloat16)
    W3b = W3.astype(jnp.bfloat16)
    W2b = W2.astype(jnp.bfloat16)
    yg = _ffn(xg, W1b, W3b, W2b, vt, ve, vlo, vhi, G)
    y0, y1 = _sc_combine_gather(yg, pos0, pos1)
    final = _combine(y0, y1, w0, w1)

    return final.reshape(B, S, H), loss2d.reshape(())
